# baseline (speedup 1.0000x reference)
import jax
import jax.numpy as jnp
from jax.experimental import pallas as pl
from jax.experimental.pallas import tpu as pltpu

NUM_EXPERTS = 8
TOP_K = 2
TILE = 128
NUM_TILES = 16
SLOTS = NUM_TILES * TILE


def _dispatch_kernel(x_ref, rw_ref, rb_ref, ept_ref, posi_ref, wts_ref):
    x = x_ref[...]
    N = x.shape[0]
    logits = jax.lax.dot_general(
        x, rw_ref[...],
        dimension_numbers=(((1,), (1,)), ((), ())),
        preferred_element_type=jnp.float32,
    ) + rb_ref[...]
    m = jnp.max(logits, axis=-1, keepdims=True)
    ex = jnp.exp(logits - m)
    p = ex / jnp.sum(ex, axis=-1, keepdims=True)
    iota = jax.lax.broadcasted_iota(jnp.int32, p.shape, 1)
    m1 = jnp.max(p, axis=-1, keepdims=True)
    i1 = jnp.min(jnp.where(p == m1, iota, NUM_EXPERTS), axis=-1, keepdims=True)
    mask1 = iota == i1
    p2 = jnp.where(mask1, -jnp.inf, p)
    m2 = jnp.max(p2, axis=-1, keepdims=True)
    i2 = jnp.min(jnp.where(p2 == m2, iota, NUM_EXPERTS), axis=-1, keepdims=True)
    mask2 = iota == i2
    denom = m1 + m2
    w1c = m1 / denom
    w2c = m2 / denom

    oh1 = mask1.astype(jnp.bfloat16)
    oh2 = mask2.astype(jnp.bfloat16)
    r0 = jax.lax.broadcasted_iota(jnp.int32, (N, N), 0)
    r1 = jax.lax.broadcasted_iota(jnp.int32, (N, N), 1)
    ltri = (r1 < r0).astype(jnp.bfloat16)
    rank1 = jnp.dot(ltri, oh1, preferred_element_type=jnp.float32)
    rank2 = jnp.dot(ltri, oh2, preferred_element_type=jnp.float32)
    cnt1 = jnp.sum(oh1.astype(jnp.float32), axis=0, keepdims=True)
    cnt2 = jnp.sum(oh2.astype(jnp.float32), axis=0, keepdims=True)
    cnt = cnt1 + cnt2
    padded = jnp.ceil(cnt * (1.0 / TILE)) * TILE
    e0 = jax.lax.broadcasted_iota(jnp.int32, (NUM_EXPERTS, NUM_EXPERTS), 0)
    e1 = jax.lax.broadcasted_iota(jnp.int32, (NUM_EXPERTS, NUM_EXPERTS), 1)
    etri = (e0 < e1).astype(jnp.float32)
    off = jax.lax.dot_general(
        padded, etri,
        dimension_numbers=(((1,), (0,)), ((), ())),
        preferred_element_type=jnp.float32,
    )

    pos1 = jnp.sum(jnp.where(mask1, off + rank1, 0.0), axis=-1, keepdims=True)
    pos2 = jnp.sum(jnp.where(mask2, off + cnt1 + rank2, 0.0),
                   axis=-1, keepdims=True)

    posi_ref[...] = (jnp.where(iota == 0, pos1, 0.0)
                     + jnp.where(iota == 1, pos2, 0.0)).astype(jnp.int32)
    wts_ref[...] = (jnp.where(iota == 0, w1c, 0.0)
                    + jnp.where(iota == 1, w2c, 0.0))

    ti = jax.lax.broadcasted_iota(jnp.int32, (8, NUM_TILES), 1) * TILE
    acc = jnp.full((8, NUM_TILES), -1, dtype=jnp.int32)
    for e in range(NUM_EXPERTS):
        off_e = jnp.sum(jnp.where(iota[0:1, :] == e, off, 0.0),
                        axis=-1, keepdims=True)
        acc = acc + jnp.where(off_e <= ti.astype(jnp.float32), 1, 0)
    ept_ref[...] = acc


def _ffn_kernel(ept_ref, x_ref, posi_ref, wts_ref, w1_ref, w2_ref, out_ref,
                xb_ref, w1b_ref, w2b_ref):
    i = pl.program_id(0)

    @pl.when(i == 0)
    def _prep():
        xb_ref[...] = x_ref[...].astype(jnp.bfloat16)
        out_ref[...] = jnp.zeros_like(out_ref)

    prev = jnp.where(i > 0, ept_ref[jnp.maximum(i - 1, 0)], -1)

    @pl.when(ept_ref[i] != prev)
    def _cast_weights():
        w1b_ref[...] = w1_ref[0].astype(jnp.bfloat16)
        w2b_ref[...] = w2_ref[0].astype(jnp.bfloat16)

    base = i * TILE
    posi = posi_ref[...]
    pos1 = posi[:, 0:1]
    pos2 = posi[:, 1:2]
    tiota = jax.lax.broadcasted_iota(jnp.int32, (posi.shape[0], TILE), 1) + base
    sel1 = tiota == pos1
    sel2 = tiota == pos2
    p2 = (sel1 | sel2).astype(jnp.bfloat16)
    xs = jax.lax.dot_general(
        p2, xb_ref[...],
        dimension_numbers=(((0,), (0,)), ((), ())),
        preferred_element_type=jnp.float32,
    ).astype(jnp.bfloat16)
    h = jnp.dot(xs, w1b_ref[...], preferred_element_type=jnp.float32)
    h = 0.5 * h * (1.0 + jax.lax.erf(h * 0.7071067811865476))
    y = jnp.dot(h.astype(jnp.bfloat16), w2b_ref[...],
                preferred_element_type=jnp.float32)
    wts = wts_ref[...]
    pw = (jnp.where(sel1, wts[:, 0:1], 0.0)
          + jnp.where(sel2, wts[:, 1:2], 0.0))
    out_ref[...] += jax.lax.dot_general(
        pw, y,
        dimension_numbers=(((1,), (0,)), ((), ())),
        preferred_element_type=jnp.float32,
    )


def kernel(x, router_w, router_b, w1, w2):
    B, T, H = x.shape
    N = B * T
    F = w1.shape[-1]
    x_flat = x.reshape(N, H)
    rb = router_b.reshape(1, NUM_EXPERTS)

    ept, posi, wts = pl.pallas_call(
        _dispatch_kernel,
        in_specs=[
            pl.BlockSpec((N, H), lambda: (0, 0)),
            pl.BlockSpec((NUM_EXPERTS, H), lambda: (0, 0)),
            pl.BlockSpec((1, NUM_EXPERTS), lambda: (0, 0)),
        ],
        out_specs=[
            pl.BlockSpec((8, NUM_TILES), lambda: (0, 0)),
            pl.BlockSpec((N, NUM_EXPERTS), lambda: (0, 0)),
            pl.BlockSpec((N, NUM_EXPERTS), lambda: (0, 0)),
        ],
        out_shape=[
            jax.ShapeDtypeStruct((8, NUM_TILES), jnp.int32),
            jax.ShapeDtypeStruct((N, NUM_EXPERTS), jnp.int32),
            jax.ShapeDtypeStruct((N, NUM_EXPERTS), jnp.float32),
        ],
    )(x_flat, router_w, rb)

    out = pl.pallas_call(
        _ffn_kernel,
        grid_spec=pltpu.PrefetchScalarGridSpec(
            num_scalar_prefetch=1,
            grid=(NUM_TILES,),
            in_specs=[
                pl.BlockSpec((N, H), lambda i, ept: (0, 0)),
                pl.BlockSpec((N, NUM_EXPERTS), lambda i, ept: (0, 0)),
                pl.BlockSpec((N, NUM_EXPERTS), lambda i, ept: (0, 0)),
                pl.BlockSpec((1, H, F), lambda i, ept: (ept[i], 0, 0)),
                pl.BlockSpec((1, F, H), lambda i, ept: (ept[i], 0, 0)),
            ],
            out_specs=pl.BlockSpec((N, H), lambda i, ept: (0, 0)),
            scratch_shapes=[
                pltpu.VMEM((N, H), jnp.bfloat16),
                pltpu.VMEM((H, F), jnp.bfloat16),
                pltpu.VMEM((F, H), jnp.bfloat16),
            ],
        ),
        out_shape=jax.ShapeDtypeStruct((N, H), jnp.float32),
    )(ept[0], x_flat, posi, wts, w1, w2)
    return out.reshape(B, T, H)


# speedup vs baseline: 1.0323x; 1.0323x over previous
import jax
import jax.numpy as jnp
from jax.experimental import pallas as pl
from jax.experimental.pallas import tpu as pltpu

NUM_EXPERTS = 8
TOP_K = 2
TILE = 128
NUM_TILES = 16
SLOTS = NUM_TILES * TILE


def _dispatch_kernel(x_ref, rw_ref, rb_ref, ept_ref, posi_ref, wts_ref):
    x = x_ref[...]
    N = x.shape[0]
    logits = jax.lax.dot_general(
        x, rw_ref[...],
        dimension_numbers=(((1,), (1,)), ((), ())),
        preferred_element_type=jnp.float32,
    ) + rb_ref[...]
    m = jnp.max(logits, axis=-1, keepdims=True)
    ex = jnp.exp(logits - m)
    p = ex / jnp.sum(ex, axis=-1, keepdims=True)
    iota = jax.lax.broadcasted_iota(jnp.int32, p.shape, 1)
    m1 = jnp.max(p, axis=-1, keepdims=True)
    i1 = jnp.min(jnp.where(p == m1, iota, NUM_EXPERTS), axis=-1, keepdims=True)
    mask1 = iota == i1
    p2 = jnp.where(mask1, -jnp.inf, p)
    m2 = jnp.max(p2, axis=-1, keepdims=True)
    i2 = jnp.min(jnp.where(p2 == m2, iota, NUM_EXPERTS), axis=-1, keepdims=True)
    mask2 = iota == i2
    denom = m1 + m2
    w1c = m1 / denom
    w2c = m2 / denom

    oh1 = mask1.astype(jnp.bfloat16)
    oh2 = mask2.astype(jnp.bfloat16)
    r0 = jax.lax.broadcasted_iota(jnp.int32, (N, N), 0)
    r1 = jax.lax.broadcasted_iota(jnp.int32, (N, N), 1)
    ltri = (r1 < r0).astype(jnp.bfloat16)
    rank1 = jnp.dot(ltri, oh1, preferred_element_type=jnp.float32)
    rank2 = jnp.dot(ltri, oh2, preferred_element_type=jnp.float32)
    cnt1 = jnp.sum(oh1.astype(jnp.float32), axis=0, keepdims=True)
    cnt2 = jnp.sum(oh2.astype(jnp.float32), axis=0, keepdims=True)
    cnt = cnt1 + cnt2
    padded = jnp.ceil(cnt * (1.0 / TILE)) * TILE
    e0 = jax.lax.broadcasted_iota(jnp.int32, (NUM_EXPERTS, NUM_EXPERTS), 0)
    e1 = jax.lax.broadcasted_iota(jnp.int32, (NUM_EXPERTS, NUM_EXPERTS), 1)
    etri = (e0 < e1).astype(jnp.float32)
    off = jax.lax.dot_general(
        padded, etri,
        dimension_numbers=(((1,), (0,)), ((), ())),
        preferred_element_type=jnp.float32,
    )

    pos1 = jnp.sum(jnp.where(mask1, off + rank1, 0.0), axis=-1, keepdims=True)
    pos2 = jnp.sum(jnp.where(mask2, off + cnt1 + rank2, 0.0),
                   axis=-1, keepdims=True)

    posi_ref[...] = (jnp.where(iota == 0, pos1, 0.0)
                     + jnp.where(iota == 1, pos2, 0.0)).astype(jnp.int32)
    wts_ref[...] = (jnp.where(iota == 0, w1c, 0.0)
                    + jnp.where(iota == 1, w2c, 0.0))

    ti = jax.lax.broadcasted_iota(jnp.int32, (8, NUM_TILES), 1) * TILE
    acc = jnp.full((8, NUM_TILES), -1, dtype=jnp.int32)
    for e in range(NUM_EXPERTS):
        off_e = jnp.sum(jnp.where(iota[0:1, :] == e, off, 0.0),
                        axis=-1, keepdims=True)
        acc = acc + jnp.where(off_e <= ti.astype(jnp.float32), 1, 0)
    ept_ref[...] = acc


def _ffn_kernel(ept_ref, x_ref, posi_ref, wts_ref, w1_ref, w2_ref, out_ref,
                xb_ref):
    i = pl.program_id(0)

    @pl.when(i == 0)
    def _prep():
        xb_ref[...] = x_ref[...].astype(jnp.bfloat16)
        out_ref[...] = jnp.zeros_like(out_ref)

    base = i * TILE
    posi = posi_ref[...]
    pos1 = posi[:, 0:1]
    pos2 = posi[:, 1:2]
    tiota = jax.lax.broadcasted_iota(jnp.int32, (posi.shape[0], TILE), 1) + base
    sel1 = tiota == pos1
    sel2 = tiota == pos2
    p2 = (sel1 | sel2).astype(jnp.bfloat16)
    xs = jax.lax.dot_general(
        p2, xb_ref[...],
        dimension_numbers=(((0,), (0,)), ((), ())),
        preferred_element_type=jnp.float32,
    ).astype(jnp.bfloat16)
    h = jnp.dot(xs, w1_ref[0].astype(jnp.bfloat16), preferred_element_type=jnp.float32)
    h = 0.5 * h * (1.0 + jax.lax.erf(h * 0.7071067811865476))
    y = jnp.dot(h.astype(jnp.bfloat16), w2_ref[0].astype(jnp.bfloat16),
                preferred_element_type=jnp.float32)
    wts = wts_ref[...]
    pw = (jnp.where(sel1, wts[:, 0:1], 0.0)
          + jnp.where(sel2, wts[:, 1:2], 0.0))
    out_ref[...] += jax.lax.dot_general(
        pw, y,
        dimension_numbers=(((1,), (0,)), ((), ())),
        preferred_element_type=jnp.float32,
    )


def kernel(x, router_w, router_b, w1, w2):
    B, T, H = x.shape
    N = B * T
    F = w1.shape[-1]
    x_flat = x.reshape(N, H)
    rb = router_b.reshape(1, NUM_EXPERTS)

    ept, posi, wts = pl.pallas_call(
        _dispatch_kernel,
        in_specs=[
            pl.BlockSpec((N, H), lambda: (0, 0)),
            pl.BlockSpec((NUM_EXPERTS, H), lambda: (0, 0)),
            pl.BlockSpec((1, NUM_EXPERTS), lambda: (0, 0)),
        ],
        out_specs=[
            pl.BlockSpec((8, NUM_TILES), lambda: (0, 0)),
            pl.BlockSpec((N, NUM_EXPERTS), lambda: (0, 0)),
            pl.BlockSpec((N, NUM_EXPERTS), lambda: (0, 0)),
        ],
        out_shape=[
            jax.ShapeDtypeStruct((8, NUM_TILES), jnp.int32),
            jax.ShapeDtypeStruct((N, NUM_EXPERTS), jnp.int32),
            jax.ShapeDtypeStruct((N, NUM_EXPERTS), jnp.float32),
        ],
    )(x_flat, router_w, rb)

    out = pl.pallas_call(
        _ffn_kernel,
        grid_spec=pltpu.PrefetchScalarGridSpec(
            num_scalar_prefetch=1,
            grid=(NUM_TILES,),
            in_specs=[
                pl.BlockSpec((N, H), lambda i, ept: (0, 0)),
                pl.BlockSpec((N, NUM_EXPERTS), lambda i, ept: (0, 0)),
                pl.BlockSpec((N, NUM_EXPERTS), lambda i, ept: (0, 0)),
                pl.BlockSpec((1, H, F), lambda i, ept: (ept[i], 0, 0)),
                pl.BlockSpec((1, F, H), lambda i, ept: (ept[i], 0, 0)),
            ],
            out_specs=pl.BlockSpec((N, H), lambda i, ept: (0, 0)),
            scratch_shapes=[
                pltpu.VMEM((N, H), jnp.bfloat16),
            ],
        ),
        out_shape=jax.ShapeDtypeStruct((N, H), jnp.float32),
    )(ept[0], x_flat, posi, wts, w1, w2)
    return out.reshape(B, T, H)


# speedup vs baseline: 1.1645x; 1.1281x over previous
import jax
import jax.numpy as jnp
from jax.experimental import pallas as pl
from jax.experimental.pallas import tpu as pltpu

NUM_EXPERTS = 8
TOP_K = 2
TILE = 128
FFN_CHUNKS = 2


def _top2_masks(logits):
    iota = jax.lax.broadcasted_iota(jnp.int32, logits.shape, 1)
    m1 = jnp.max(logits, axis=-1, keepdims=True)
    i1 = jnp.min(jnp.where(logits == m1, iota, NUM_EXPERTS), axis=-1,
                 keepdims=True)
    mask1 = iota == i1
    l2 = jnp.where(mask1, -jnp.inf, logits)
    m2 = jnp.max(l2, axis=-1, keepdims=True)
    i2 = jnp.min(jnp.where(l2 == m2, iota, NUM_EXPERTS), axis=-1,
                 keepdims=True)
    mask2 = iota == i2
    return mask1, mask2, m1, m2, iota


def _expert_offsets(ntiles):
    e0 = jax.lax.broadcasted_iota(jnp.int32, (NUM_EXPERTS, NUM_EXPERTS), 0)
    e1 = jax.lax.broadcasted_iota(jnp.int32, (NUM_EXPERTS, NUM_EXPERTS), 1)
    etri = (e0 < e1).astype(jnp.float32)
    return jax.lax.dot_general(
        ntiles * TILE, etri,
        dimension_numbers=(((1,), (0,)), ((), ())),
        preferred_element_type=jnp.float32,
    )


def _counts_kernel(x_ref, rw_ref, rb_ref, nt_ref):
    logits = jax.lax.dot_general(
        x_ref[...], rw_ref[...],
        dimension_numbers=(((1,), (1,)), ((), ())),
        preferred_element_type=jnp.float32,
    ) + rb_ref[...]
    mask1, mask2, _, _, _ = _top2_masks(logits)
    cnt = (jnp.sum(mask1.astype(jnp.float32), axis=0, keepdims=True)
           + jnp.sum(mask2.astype(jnp.float32), axis=0, keepdims=True))
    ntiles = jnp.ceil(cnt * (1.0 / TILE))
    off = _expert_offsets(ntiles)
    row = jnp.concatenate([ntiles, off], axis=1)
    nt_ref[...] = jnp.broadcast_to(row, (8, 2 * NUM_EXPERTS)).astype(jnp.int32)


def _ffn_kernel(sc_ref, x_ref, rw_ref, rb_ref, w1_ref, w2_ref, out_ref,
                xb_ref, posi_ref, wts_ref):
    e = pl.program_id(0)
    c = pl.program_id(1)
    step = e * FFN_CHUNKS + c

    @pl.when(step == 0)
    def _dispatch():
        x = x_ref[...]
        xb_ref[...] = x.astype(jnp.bfloat16)
        out_ref[...] = jnp.zeros_like(out_ref)
        N = x.shape[0]
        logits = jax.lax.dot_general(
            x, rw_ref[...],
            dimension_numbers=(((1,), (1,)), ((), ())),
            preferred_element_type=jnp.float32,
        ) + rb_ref[...]
        mask1, mask2, m1, m2, iota = _top2_masks(logits)
        mx = jnp.maximum(m1, m2)
        ex1 = jnp.exp(m1 - mx)
        ex2 = jnp.exp(m2 - mx)
        denom = ex1 + ex2
        w1c = ex1 / denom
        w2c = ex2 / denom
        oh1 = mask1.astype(jnp.bfloat16)
        oh2 = mask2.astype(jnp.bfloat16)
        r0 = jax.lax.broadcasted_iota(jnp.int32, (N, N), 0)
        r1 = jax.lax.broadcasted_iota(jnp.int32, (N, N), 1)
        ltri = (r1 < r0).astype(jnp.bfloat16)
        rank1 = jnp.dot(ltri, oh1, preferred_element_type=jnp.float32)
        rank2 = jnp.dot(ltri, oh2, preferred_element_type=jnp.float32)
        cnt1 = jnp.sum(oh1.astype(jnp.float32), axis=0, keepdims=True)
        cnt2 = jnp.sum(oh2.astype(jnp.float32), axis=0, keepdims=True)
        ntiles = jnp.ceil((cnt1 + cnt2) * (1.0 / TILE))
        off = _expert_offsets(ntiles)
        pos1 = jnp.sum(jnp.where(mask1, off + rank1, 0.0), axis=-1,
                       keepdims=True)
        pos2 = jnp.sum(jnp.where(mask2, off + cnt1 + rank2, 0.0), axis=-1,
                       keepdims=True)
        posi_ref[...] = (jnp.where(iota == 0, pos1, 0.0)
                         + jnp.where(iota == 1, pos2, 0.0)).astype(jnp.int32)
        wts_ref[...] = (jnp.where(iota == 0, w1c, 0.0)
                        + jnp.where(iota == 1, w2c, 0.0))

    nt_e = sc_ref[e]
    off_e = sc_ref[NUM_EXPERTS + e]
    w1b = w1_ref[0].astype(jnp.bfloat16)
    w2b = w2_ref[0].astype(jnp.bfloat16)
    posi = posi_ref[...]
    pos1 = posi[:, 0:1]
    pos2 = posi[:, 1:2]
    wts = wts_ref[...]
    tiota = jax.lax.broadcasted_iota(jnp.int32, (posi.shape[0], TILE), 1)

    def _tile(j, carry):
        base = off_e + j * TILE
        t = tiota + base
        sel1 = t == pos1
        sel2 = t == pos2
        p2 = (sel1 | sel2).astype(jnp.bfloat16)
        xs = jax.lax.dot_general(
            p2, xb_ref[...],
            dimension_numbers=(((0,), (0,)), ((), ())),
            preferred_element_type=jnp.float32,
        ).astype(jnp.bfloat16)
        h = jnp.dot(xs, w1b, preferred_element_type=jnp.float32)
        h = 0.5 * h * (1.0 + jax.lax.erf(h * 0.7071067811865476))
        y = jnp.dot(h.astype(jnp.bfloat16), w2b,
                    preferred_element_type=jnp.float32)
        pw = (jnp.where(sel1, wts[:, 0:1], 0.0)
              + jnp.where(sel2, wts[:, 1:2], 0.0))
        out_ref[...] += jax.lax.dot_general(
            pw, y,
            dimension_numbers=(((1,), (0,)), ((), ())),
            preferred_element_type=jnp.float32,
        )
        return carry

    jax.lax.fori_loop(0, nt_e, _tile, 0)


def kernel(x, router_w, router_b, w1, w2):
    B, T, H = x.shape
    N = B * T
    F = w1.shape[-1]
    Fc = F // FFN_CHUNKS
    x_flat = x.reshape(N, H)
    rb = router_b.reshape(1, NUM_EXPERTS)

    ntoff = pl.pallas_call(
        _counts_kernel,
        in_specs=[
            pl.BlockSpec((N, H), lambda: (0, 0)),
            pl.BlockSpec((NUM_EXPERTS, H), lambda: (0, 0)),
            pl.BlockSpec((1, NUM_EXPERTS), lambda: (0, 0)),
        ],
        out_specs=pl.BlockSpec((8, 2 * NUM_EXPERTS), lambda: (0, 0)),
        out_shape=jax.ShapeDtypeStruct((8, 2 * NUM_EXPERTS), jnp.int32),
    )(x_flat, router_w, rb)

    out = pl.pallas_call(
        _ffn_kernel,
        grid_spec=pltpu.PrefetchScalarGridSpec(
            num_scalar_prefetch=1,
            grid=(NUM_EXPERTS, FFN_CHUNKS),
            in_specs=[
                pl.BlockSpec((N, H), lambda e, c, sc: (0, 0)),
                pl.BlockSpec((NUM_EXPERTS, H), lambda e, c, sc: (0, 0)),
                pl.BlockSpec((1, NUM_EXPERTS), lambda e, c, sc: (0, 0)),
                pl.BlockSpec((1, H, Fc), lambda e, c, sc: (e, 0, c)),
                pl.BlockSpec((1, Fc, H), lambda e, c, sc: (e, c, 0)),
            ],
            out_specs=pl.BlockSpec((N, H), lambda e, c, sc: (0, 0)),
            scratch_shapes=[
                pltpu.VMEM((N, H), jnp.bfloat16),
                pltpu.VMEM((N, NUM_EXPERTS), jnp.int32),
                pltpu.VMEM((N, NUM_EXPERTS), jnp.float32),
            ],
        ),
        out_shape=jax.ShapeDtypeStruct((N, H), jnp.float32),
    )(ntoff[0], x_flat, router_w, rb, w1, w2)
    return out.reshape(B, T, H)


# speedup vs baseline: 1.1819x; 1.0149x over previous
import jax
import jax.numpy as jnp
from jax.experimental import pallas as pl
from jax.experimental.pallas import tpu as pltpu

NUM_EXPERTS = 8
TOP_K = 2
TILE = 128
SLOTS = 2048
FFN_CHUNKS = 2


def _top2_masks(logits):
    iota = jax.lax.broadcasted_iota(jnp.int32, logits.shape, 1)
    m1 = jnp.max(logits, axis=-1, keepdims=True)
    i1 = jnp.min(jnp.where(logits == m1, iota, NUM_EXPERTS), axis=-1,
                 keepdims=True)
    mask1 = iota == i1
    l2 = jnp.where(mask1, -jnp.inf, logits)
    m2 = jnp.max(l2, axis=-1, keepdims=True)
    i2 = jnp.min(jnp.where(l2 == m2, iota, NUM_EXPERTS), axis=-1,
                 keepdims=True)
    mask2 = iota == i2
    return mask1, mask2, m1, m2, iota


def _expert_offsets(ntiles):
    e0 = jax.lax.broadcasted_iota(jnp.int32, (NUM_EXPERTS, NUM_EXPERTS), 0)
    e1 = jax.lax.broadcasted_iota(jnp.int32, (NUM_EXPERTS, NUM_EXPERTS), 1)
    etri = (e0 < e1).astype(jnp.float32)
    return jax.lax.dot_general(
        ntiles * TILE, etri,
        dimension_numbers=(((1,), (0,)), ((), ())),
        preferred_element_type=jnp.float32,
    )


def _counts_kernel(x_ref, rw_ref, rb_ref, nt_ref):
    logits = jax.lax.dot_general(
        x_ref[...], rw_ref[...],
        dimension_numbers=(((1,), (1,)), ((), ())),
        preferred_element_type=jnp.float32,
    ) + rb_ref[...]
    mask1, mask2, _, _, _ = _top2_masks(logits)
    cnt = (jnp.sum(mask1.astype(jnp.float32), axis=0, keepdims=True)
           + jnp.sum(mask2.astype(jnp.float32), axis=0, keepdims=True))
    ntiles = jnp.ceil(cnt * (1.0 / TILE))
    off = _expert_offsets(ntiles)
    row = jnp.concatenate([ntiles, off], axis=1)
    nt_ref[...] = jnp.broadcast_to(row, (8, 2 * NUM_EXPERTS)).astype(jnp.int32)


def _ffn_kernel(sc_ref, x_ref, rw_ref, rb_ref, w1_ref, w2_ref, out_ref,
                xb_ref, posi_ref, wts_ref, y_ref):
    e = pl.program_id(0)
    c = pl.program_id(1)
    step = e * FFN_CHUNKS + c

    @pl.when(step == 0)
    def _dispatch():
        x = x_ref[...]
        xb_ref[...] = x.astype(jnp.bfloat16)
        y_ref[...] = jnp.zeros_like(y_ref)
        N = x.shape[0]
        logits = jax.lax.dot_general(
            x, rw_ref[...],
            dimension_numbers=(((1,), (1,)), ((), ())),
            preferred_element_type=jnp.float32,
        ) + rb_ref[...]
        mask1, mask2, m1, m2, iota = _top2_masks(logits)
        mx = jnp.maximum(m1, m2)
        ex1 = jnp.exp(m1 - mx)
        ex2 = jnp.exp(m2 - mx)
        denom = ex1 + ex2
        w1c = ex1 / denom
        w2c = ex2 / denom
        oh1 = mask1.astype(jnp.bfloat16)
        oh2 = mask2.astype(jnp.bfloat16)
        r0 = jax.lax.broadcasted_iota(jnp.int32, (N, N), 0)
        r1 = jax.lax.broadcasted_iota(jnp.int32, (N, N), 1)
        ltri = (r1 < r0).astype(jnp.bfloat16)
        rank1 = jnp.dot(ltri, oh1, preferred_element_type=jnp.float32)
        rank2 = jnp.dot(ltri, oh2, preferred_element_type=jnp.float32)
        cnt1 = jnp.sum(oh1.astype(jnp.float32), axis=0, keepdims=True)
        cnt2 = jnp.sum(oh2.astype(jnp.float32), axis=0, keepdims=True)
        ntiles = jnp.ceil((cnt1 + cnt2) * (1.0 / TILE))
        off = _expert_offsets(ntiles)
        pos1 = jnp.sum(jnp.where(mask1, off + rank1, 0.0), axis=-1,
                       keepdims=True)
        pos2 = jnp.sum(jnp.where(mask2, off + cnt1 + rank2, 0.0), axis=-1,
                       keepdims=True)
        posi_ref[...] = (jnp.where(iota == 0, pos1, 0.0)
                         + jnp.where(iota == 1, pos2, 0.0)).astype(jnp.int32)
        wts_ref[...] = (jnp.where(iota == 0, w1c, 0.0)
                        + jnp.where(iota == 1, w2c, 0.0))

    nt_e = sc_ref[e]
    off_e = sc_ref[NUM_EXPERTS + e]
    w1b = w1_ref[0].astype(jnp.bfloat16)
    w2b = w2_ref[0].astype(jnp.bfloat16)
    posi = posi_ref[...]
    pos1 = posi[:, 0:1]
    pos2 = posi[:, 1:2]
    wts = wts_ref[...]
    tiota = jax.lax.broadcasted_iota(jnp.int32, (posi.shape[0], TILE), 1)

    def _tile(j, carry):
        base = off_e + j * TILE
        t = tiota + base
        sel1 = t == pos1
        sel2 = t == pos2
        p2 = (sel1 | sel2).astype(jnp.bfloat16)
        xs = jax.lax.dot_general(
            p2, xb_ref[...],
            dimension_numbers=(((0,), (0,)), ((), ())),
            preferred_element_type=jnp.float32,
        ).astype(jnp.bfloat16)
        h = jnp.dot(xs, w1b, preferred_element_type=jnp.float32)
        h = 0.5 * h * (1.0 + jax.lax.erf(h * 0.7071067811865476))
        y = jnp.dot(h.astype(jnp.bfloat16), w2b,
                    preferred_element_type=jnp.float32)
        bh = pl.multiple_of(base, TILE)

        @pl.when(c == 0)
        def _store():
            y_ref[pl.ds(bh, TILE), :] = y

        @pl.when(c != 0)
        def _accum():
            y_ref[pl.ds(bh, TILE), :] += y

        return carry

    jax.lax.fori_loop(0, nt_e, _tile, 0)

    @pl.when(step == NUM_EXPERTS * FFN_CHUNKS - 1)
    def _combine():
        siota = jax.lax.broadcasted_iota(jnp.int32, (posi.shape[0], SLOTS), 1)
        pw = (jnp.where(siota == pos1, wts[:, 0:1], 0.0)
              + jnp.where(siota == pos2, wts[:, 1:2], 0.0))
        out_ref[...] = jax.lax.dot_general(
            pw.astype(jnp.bfloat16), y_ref[...].astype(jnp.bfloat16),
            dimension_numbers=(((1,), (0,)), ((), ())),
            preferred_element_type=jnp.float32,
        )


def kernel(x, router_w, router_b, w1, w2):
    B, T, H = x.shape
    N = B * T
    F = w1.shape[-1]
    Fc = F // FFN_CHUNKS
    x_flat = x.reshape(N, H)
    rb = router_b.reshape(1, NUM_EXPERTS)

    ntoff = pl.pallas_call(
        _counts_kernel,
        in_specs=[
            pl.BlockSpec((N, H), lambda: (0, 0)),
            pl.BlockSpec((NUM_EXPERTS, H), lambda: (0, 0)),
            pl.BlockSpec((1, NUM_EXPERTS), lambda: (0, 0)),
        ],
        out_specs=pl.BlockSpec((8, 2 * NUM_EXPERTS), lambda: (0, 0)),
        out_shape=jax.ShapeDtypeStruct((8, 2 * NUM_EXPERTS), jnp.int32),
    )(x_flat, router_w, rb)

    out = pl.pallas_call(
        _ffn_kernel,
        grid_spec=pltpu.PrefetchScalarGridSpec(
            num_scalar_prefetch=1,
            grid=(NUM_EXPERTS, FFN_CHUNKS),
            in_specs=[
                pl.BlockSpec((N, H), lambda e, c, sc: (0, 0)),
                pl.BlockSpec((NUM_EXPERTS, H), lambda e, c, sc: (0, 0)),
                pl.BlockSpec((1, NUM_EXPERTS), lambda e, c, sc: (0, 0)),
                pl.BlockSpec((1, H, Fc), lambda e, c, sc: (e, 0, c)),
                pl.BlockSpec((1, Fc, H), lambda e, c, sc: (e, c, 0)),
            ],
            out_specs=pl.BlockSpec((N, H), lambda e, c, sc: (0, 0)),
            scratch_shapes=[
                pltpu.VMEM((N, H), jnp.bfloat16),
                pltpu.VMEM((N, NUM_EXPERTS), jnp.int32),
                pltpu.VMEM((N, NUM_EXPERTS), jnp.float32),
                pltpu.VMEM((SLOTS, H), jnp.float32),
            ],
        ),
        out_shape=jax.ShapeDtypeStruct((N, H), jnp.float32),
    )(ntoff[0], x_flat, router_w, rb, w1, w2)
    return out.reshape(B, T, H)


# speedup vs baseline: 1.2876x; 1.0895x over previous
import jax
import jax.numpy as jnp
from jax.experimental import pallas as pl
from jax.experimental.pallas import tpu as pltpu

NUM_EXPERTS = 8
TOP_K = 2
TILE = 128
SLOTS = 2048
FFN_CHUNKS = 1


def _top2_masks(logits):
    iota = jax.lax.broadcasted_iota(jnp.int32, logits.shape, 1)
    m1 = jnp.max(logits, axis=-1, keepdims=True)
    i1 = jnp.min(jnp.where(logits == m1, iota, NUM_EXPERTS), axis=-1,
                 keepdims=True)
    mask1 = iota == i1
    l2 = jnp.where(mask1, -jnp.inf, logits)
    m2 = jnp.max(l2, axis=-1, keepdims=True)
    i2 = jnp.min(jnp.where(l2 == m2, iota, NUM_EXPERTS), axis=-1,
                 keepdims=True)
    mask2 = iota == i2
    return mask1, mask2, m1, m2, iota


def _expert_offsets(ntiles):
    e0 = jax.lax.broadcasted_iota(jnp.int32, (NUM_EXPERTS, NUM_EXPERTS), 0)
    e1 = jax.lax.broadcasted_iota(jnp.int32, (NUM_EXPERTS, NUM_EXPERTS), 1)
    etri = (e0 < e1).astype(jnp.float32)
    return jax.lax.dot_general(
        ntiles * TILE, etri,
        dimension_numbers=(((1,), (0,)), ((), ())),
        preferred_element_type=jnp.float32,
    )


def _counts_kernel(x_ref, rw_ref, rb_ref, nt_ref):
    logits = jax.lax.dot_general(
        x_ref[...], rw_ref[...],
        dimension_numbers=(((1,), (1,)), ((), ())),
        preferred_element_type=jnp.float32,
    ) + rb_ref[...]
    mask1, mask2, _, _, _ = _top2_masks(logits)
    cnt = (jnp.sum(mask1.astype(jnp.float32), axis=0, keepdims=True)
           + jnp.sum(mask2.astype(jnp.float32), axis=0, keepdims=True))
    ntiles = jnp.ceil(cnt * (1.0 / TILE))
    off = _expert_offsets(ntiles)
    row = jnp.concatenate([ntiles, off], axis=1)
    nt_ref[...] = jnp.broadcast_to(row, (8, 2 * NUM_EXPERTS)).astype(jnp.int32)


def _ffn_kernel(sc_ref, x_ref, rw_ref, rb_ref, w1_ref, w2_ref, out_ref,
                xb_ref, posi_ref, wts_ref, y_ref, w1b_ref, w2b_ref):
    e = pl.program_id(0)
    c = pl.program_id(1)
    step = e * FFN_CHUNKS + c

    @pl.when(step == 0)
    def _dispatch():
        x = x_ref[...]
        xb_ref[...] = x.astype(jnp.bfloat16)
        y_ref[...] = jnp.zeros_like(y_ref)
        N = x.shape[0]
        logits = jax.lax.dot_general(
            x, rw_ref[...],
            dimension_numbers=(((1,), (1,)), ((), ())),
            preferred_element_type=jnp.float32,
        ) + rb_ref[...]
        mask1, mask2, m1, m2, iota = _top2_masks(logits)
        mx = jnp.maximum(m1, m2)
        ex1 = jnp.exp(m1 - mx)
        ex2 = jnp.exp(m2 - mx)
        denom = ex1 + ex2
        w1c = ex1 / denom
        w2c = ex2 / denom
        oh1 = mask1.astype(jnp.bfloat16)
        oh2 = mask2.astype(jnp.bfloat16)
        KC = 128
        rank1 = jnp.zeros((N, NUM_EXPERTS), jnp.float32)
        rank2 = jnp.zeros((N, NUM_EXPERTS), jnp.float32)
        r0 = jax.lax.broadcasted_iota(jnp.int32, (N, KC), 0)
        r1 = jax.lax.broadcasted_iota(jnp.int32, (N, KC), 1)
        for kk in range(N // KC):
            ltri_k = (r1 + kk * KC < r0).astype(jnp.bfloat16)
            rank1 = rank1 + jnp.dot(ltri_k, oh1[kk * KC:(kk + 1) * KC],
                                    preferred_element_type=jnp.float32)
            rank2 = rank2 + jnp.dot(ltri_k, oh2[kk * KC:(kk + 1) * KC],
                                    preferred_element_type=jnp.float32)
        cnt1 = jnp.sum(oh1.astype(jnp.float32), axis=0, keepdims=True)
        cnt2 = jnp.sum(oh2.astype(jnp.float32), axis=0, keepdims=True)
        ntiles = jnp.ceil((cnt1 + cnt2) * (1.0 / TILE))
        off = _expert_offsets(ntiles)
        pos1 = jnp.sum(jnp.where(mask1, off + rank1, 0.0), axis=-1,
                       keepdims=True)
        pos2 = jnp.sum(jnp.where(mask2, off + cnt1 + rank2, 0.0), axis=-1,
                       keepdims=True)
        posi_ref[...] = (jnp.where(iota == 0, pos1, 0.0)
                         + jnp.where(iota == 1, pos2, 0.0)).astype(jnp.int32)
        wts_ref[...] = (jnp.where(iota == 0, w1c, 0.0)
                        + jnp.where(iota == 1, w2c, 0.0))

    nt_e = sc_ref[e]
    off_e = sc_ref[NUM_EXPERTS + e]
    w1b_ref[...] = w1_ref[0].astype(jnp.bfloat16)
    w2b_ref[...] = w2_ref[0].astype(jnp.bfloat16)
    posi = posi_ref[...]
    pos1 = posi[:, 0:1]
    pos2 = posi[:, 1:2]
    wts = wts_ref[...]
    tiota = jax.lax.broadcasted_iota(jnp.int32, (posi.shape[0], TILE), 1)

    def _tile(j, carry):
        base = off_e + j * TILE
        t = tiota + base
        sel1 = t == pos1
        sel2 = t == pos2
        p2 = (sel1 | sel2).astype(jnp.bfloat16)
        xs = jax.lax.dot_general(
            p2, xb_ref[...],
            dimension_numbers=(((0,), (0,)), ((), ())),
            preferred_element_type=jnp.float32,
        ).astype(jnp.bfloat16)
        h = jnp.dot(xs, w1b_ref[...], preferred_element_type=jnp.float32)
        h = 0.5 * h * (1.0 + jax.lax.erf(h * 0.7071067811865476))
        y = jnp.dot(h.astype(jnp.bfloat16), w2b_ref[...],
                    preferred_element_type=jnp.float32)
        bh = pl.multiple_of(base, TILE)

        @pl.when(c == 0)
        def _store():
            y_ref[pl.ds(bh, TILE), :] = y.astype(jnp.bfloat16)

        @pl.when(c != 0)
        def _accum():
            y_ref[pl.ds(bh, TILE), :] += y.astype(jnp.bfloat16)

        return carry

    jax.lax.fori_loop(0, nt_e, _tile, 0)

    @pl.when(step == NUM_EXPERTS * FFN_CHUNKS - 1)
    def _combine():
        CHUNK = 512
        siota = jax.lax.broadcasted_iota(jnp.int32, (posi.shape[0], CHUNK), 1)
        acc = jnp.zeros(out_ref.shape, jnp.float32)
        for k in range(SLOTS // CHUNK):
            sk = siota + k * CHUNK
            pw = (jnp.where(sk == pos1, wts[:, 0:1], 0.0)
                  + jnp.where(sk == pos2, wts[:, 1:2], 0.0))
            acc = acc + jax.lax.dot_general(
                pw.astype(jnp.bfloat16), y_ref[pl.ds(k * CHUNK, CHUNK), :],
                dimension_numbers=(((1,), (0,)), ((), ())),
                preferred_element_type=jnp.float32,
            )
        out_ref[...] = acc


def kernel(x, router_w, router_b, w1, w2):
    B, T, H = x.shape
    N = B * T
    F = w1.shape[-1]
    Fc = F // FFN_CHUNKS
    x_flat = x.reshape(N, H)
    rb = router_b.reshape(1, NUM_EXPERTS)

    ntoff = pl.pallas_call(
        _counts_kernel,
        in_specs=[
            pl.BlockSpec((N, H), lambda: (0, 0)),
            pl.BlockSpec((NUM_EXPERTS, H), lambda: (0, 0)),
            pl.BlockSpec((1, NUM_EXPERTS), lambda: (0, 0)),
        ],
        out_specs=pl.BlockSpec((8, 2 * NUM_EXPERTS), lambda: (0, 0)),
        out_shape=jax.ShapeDtypeStruct((8, 2 * NUM_EXPERTS), jnp.int32),
    )(x_flat, router_w, rb)

    out = pl.pallas_call(
        _ffn_kernel,
        grid_spec=pltpu.PrefetchScalarGridSpec(
            num_scalar_prefetch=1,
            grid=(NUM_EXPERTS, FFN_CHUNKS),
            in_specs=[
                pl.BlockSpec((N, H), lambda e, c, sc: (0, 0)),
                pl.BlockSpec((NUM_EXPERTS, H), lambda e, c, sc: (0, 0)),
                pl.BlockSpec((1, NUM_EXPERTS), lambda e, c, sc: (0, 0)),
                pl.BlockSpec((1, H, Fc), lambda e, c, sc: (e, 0, c)),
                pl.BlockSpec((1, Fc, H), lambda e, c, sc: (e, c, 0)),
            ],
            out_specs=pl.BlockSpec((N, H), lambda e, c, sc: (0, 0)),
            scratch_shapes=[
                pltpu.VMEM((N, H), jnp.bfloat16),
                pltpu.VMEM((N, NUM_EXPERTS), jnp.int32),
                pltpu.VMEM((N, NUM_EXPERTS), jnp.float32),
                pltpu.VMEM((SLOTS, H), jnp.bfloat16),
                pltpu.VMEM((H, F), jnp.bfloat16),
                pltpu.VMEM((F, H), jnp.bfloat16),
            ],
        ),
        out_shape=jax.ShapeDtypeStruct((N, H), jnp.float32),
    )(ntoff[0], x_flat, router_w, rb, w1, w2)
    return out.reshape(B, T, H)


# speedup vs baseline: 1.3873x; 1.0775x over previous
import functools

import jax
import jax.numpy as jnp
from jax.experimental import pallas as pl
from jax.experimental.pallas import tpu as pltpu

NUM_EXPERTS = 8
TOP_K = 2


def _moe_kernel(x_ref, rw_ref, rb_ref, w1_ref, w2_ref, out_ref, w_scratch):
    e = pl.program_id(0)

    @pl.when(e == 0)
    def _router():
        x = x_ref[...]
        logits = jax.lax.dot_general(
            x, rw_ref[...],
            dimension_numbers=(((1,), (1,)), ((), ())),
            preferred_element_type=jnp.float32,
        ) + rb_ref[...]
        m = jnp.max(logits, axis=-1, keepdims=True)
        ex = jnp.exp(logits - m)
        p = ex / jnp.sum(ex, axis=-1, keepdims=True)
        iota = jax.lax.broadcasted_iota(jnp.int32, p.shape, 1)
        m1 = jnp.max(p, axis=-1, keepdims=True)
        i1 = jnp.min(jnp.where(p == m1, iota, NUM_EXPERTS), axis=-1, keepdims=True)
        mask1 = iota == i1
        p2 = jnp.where(mask1, -jnp.inf, p)
        m2 = jnp.max(p2, axis=-1, keepdims=True)
        i2 = jnp.min(jnp.where(p2 == m2, iota, NUM_EXPERTS), axis=-1, keepdims=True)
        mask2 = iota == i2
        denom = m1 + m2
        w_scratch[...] = jnp.where(mask1 | mask2, p / denom, 0.0)

    @pl.when(e == 0)
    def _zero():
        out_ref[...] = jnp.zeros_like(out_ref)

    w_all = w_scratch[...]
    eiota = jax.lax.broadcasted_iota(jnp.int32, w_all.shape, 1)
    wcol = jnp.sum(jnp.where(eiota == e, w_all, 0.0), axis=-1, keepdims=True)
    w1b = w1_ref[0].astype(jnp.bfloat16)
    w2b = w2_ref[0].astype(jnp.bfloat16)
    MC = 256
    for mk in range(2):
        x = x_ref[pl.ds(mk * MC, MC), :].astype(jnp.bfloat16)
        h = jnp.dot(x, w1b, preferred_element_type=jnp.float32)
        h = 0.5 * h * (1.0 + jax.lax.erf(h * 0.7071067811865476))
        y = jnp.dot(h.astype(jnp.bfloat16), w2b,
                    preferred_element_type=jnp.float32)
        out_ref[pl.ds(mk * MC, MC), :] += wcol[mk * MC:(mk + 1) * MC] * y


def kernel(x, router_w, router_b, w1, w2):
    B, T, H = x.shape
    N = B * T
    F = w1.shape[-1]
    x_flat = x.reshape(N, H)
    rb = router_b.reshape(1, NUM_EXPERTS)

    out = pl.pallas_call(
        _moe_kernel,
        grid=(NUM_EXPERTS,),
        in_specs=[
            pl.BlockSpec((N, H), lambda e: (0, 0)),
            pl.BlockSpec((NUM_EXPERTS, H), lambda e: (0, 0)),
            pl.BlockSpec((1, NUM_EXPERTS), lambda e: (0, 0)),
            pl.BlockSpec((1, H, F), lambda e: (e, 0, 0)),
            pl.BlockSpec((1, F, H), lambda e: (e, 0, 0)),
        ],
        out_specs=pl.BlockSpec((N, H), lambda e: (0, 0)),
        out_shape=jax.ShapeDtypeStruct((N, H), jnp.float32),
        scratch_shapes=[pltpu.VMEM((N, NUM_EXPERTS), jnp.float32)],
    )(x_flat, router_w, rb, w1, w2)
    return out.reshape(B, T, H)


# speedup vs baseline: 1.3892x; 1.0013x over previous
import functools

import jax
import jax.numpy as jnp
from jax.experimental import pallas as pl
from jax.experimental.pallas import tpu as pltpu

NUM_EXPERTS = 8
TOP_K = 2


def _moe_kernel(x_ref, rw_ref, rb_ref, w1_ref, w2_ref, out_ref, w_scratch):
    e = pl.program_id(0)

    @pl.when(e == 0)
    def _router():
        x = x_ref[...]
        logits = jax.lax.dot_general(
            x, rw_ref[...],
            dimension_numbers=(((1,), (1,)), ((), ())),
            preferred_element_type=jnp.float32,
        ) + rb_ref[...]
        m = jnp.max(logits, axis=-1, keepdims=True)
        ex = jnp.exp(logits - m)
        p = ex / jnp.sum(ex, axis=-1, keepdims=True)
        iota = jax.lax.broadcasted_iota(jnp.int32, p.shape, 1)
        m1 = jnp.max(p, axis=-1, keepdims=True)
        i1 = jnp.min(jnp.where(p == m1, iota, NUM_EXPERTS), axis=-1, keepdims=True)
        mask1 = iota == i1
        p2 = jnp.where(mask1, -jnp.inf, p)
        m2 = jnp.max(p2, axis=-1, keepdims=True)
        i2 = jnp.min(jnp.where(p2 == m2, iota, NUM_EXPERTS), axis=-1, keepdims=True)
        mask2 = iota == i2
        denom = m1 + m2
        w_scratch[...] = jnp.where(mask1 | mask2, p / denom, 0.0)

    @pl.when(e == 0)
    def _zero():
        out_ref[...] = jnp.zeros_like(out_ref)

    x = x_ref[...].astype(jnp.bfloat16)
    h = jnp.dot(x, w1_ref[0].astype(jnp.bfloat16),
                preferred_element_type=jnp.float32)
    h = 0.5 * h * (1.0 + jax.lax.erf(h * 0.7071067811865476))
    y = jnp.dot(h.astype(jnp.bfloat16), w2_ref[0].astype(jnp.bfloat16),
                preferred_element_type=jnp.float32)
    w_all = w_scratch[...]
    eiota = jax.lax.broadcasted_iota(jnp.int32, w_all.shape, 1)
    wcol = jnp.sum(jnp.where(eiota == e, w_all, 0.0), axis=-1, keepdims=True)
    out_ref[...] += wcol * y


def kernel(x, router_w, router_b, w1, w2):
    B, T, H = x.shape
    N = B * T
    F = w1.shape[-1]
    x_flat = x.reshape(N, H)
    rb = router_b.reshape(1, NUM_EXPERTS)

    out = pl.pallas_call(
        _moe_kernel,
        grid=(NUM_EXPERTS,),
        in_specs=[
            pl.BlockSpec((N, H), lambda e: (0, 0)),
            pl.BlockSpec((NUM_EXPERTS, H), lambda e: (0, 0)),
            pl.BlockSpec((1, NUM_EXPERTS), lambda e: (0, 0)),
            pl.BlockSpec((1, H, F), lambda e: (e, 0, 0)),
            pl.BlockSpec((1, F, H), lambda e: (e, 0, 0)),
        ],
        out_specs=pl.BlockSpec((N, H), lambda e: (0, 0)),
        out_shape=jax.ShapeDtypeStruct((N, H), jnp.float32),
        scratch_shapes=[pltpu.VMEM((N, NUM_EXPERTS), jnp.float32)],
    )(x_flat, router_w, rb, w1, w2)
    return out.reshape(B, T, H)


# speedup vs baseline: 1.3920x; 1.0020x over previous
import jax
import jax.numpy as jnp
from jax.experimental import pallas as pl
from jax.experimental.pallas import tpu as pltpu

NUM_EXPERTS = 8
TOP_K = 2


def _moe_kernel(x_ref, rw_ref, rb_ref, w1_ref, w2_ref, out_ref, w_scratch):
    e = pl.program_id(0)

    @pl.when(e == 0)
    def _router():
        x = x_ref[...]
        logits = jax.lax.dot_general(
            x, rw_ref[...],
            dimension_numbers=(((1,), (1,)), ((), ())),
            preferred_element_type=jnp.float32,
        ) + rb_ref[...]
        m = jnp.max(logits, axis=-1, keepdims=True)
        ex = jnp.exp(logits - m)
        p = ex / jnp.sum(ex, axis=-1, keepdims=True)
        iota = jax.lax.broadcasted_iota(jnp.int32, p.shape, 1)
        m1 = jnp.max(p, axis=-1, keepdims=True)
        i1 = jnp.min(jnp.where(p == m1, iota, NUM_EXPERTS), axis=-1, keepdims=True)
        mask1 = iota == i1
        p2 = jnp.where(mask1, -jnp.inf, p)
        m2 = jnp.max(p2, axis=-1, keepdims=True)
        i2 = jnp.min(jnp.where(p2 == m2, iota, NUM_EXPERTS), axis=-1, keepdims=True)
        mask2 = iota == i2
        denom = m1 + m2
        w_scratch[...] = jnp.where(mask1 | mask2, p / denom, 0.0)

    @pl.when(e == 0)
    def _zero():
        out_ref[...] = jnp.zeros_like(out_ref)

    x = x_ref[...].astype(jnp.bfloat16)
    h = jnp.dot(x, w1_ref[0].astype(jnp.bfloat16),
                preferred_element_type=jnp.float32)
    h = 0.5 * h * (1.0 + jax.lax.erf(h * 0.7071067811865476))
    y = jnp.dot(h.astype(jnp.bfloat16), w2_ref[0].astype(jnp.bfloat16),
                preferred_element_type=jnp.float32)
    w_all = w_scratch[...]
    eiota = jax.lax.broadcasted_iota(jnp.int32, w_all.shape, 1)
    wcol = jnp.sum(jnp.where(eiota == e, w_all, 0.0), axis=-1, keepdims=True)
    out_ref[...] += wcol * y


def kernel(x, router_w, router_b, w1, w2):
    B, T, H = x.shape
    N = B * T
    F = w1.shape[-1]
    x_flat = x.reshape(N, H)
    rb = router_b.reshape(1, NUM_EXPERTS)

    out = pl.pallas_call(
        _moe_kernel,
        grid=(NUM_EXPERTS,),
        in_specs=[
            pl.BlockSpec((N, H), lambda e: (0, 0)),
            pl.BlockSpec((NUM_EXPERTS, H), lambda e: (0, 0)),
            pl.BlockSpec((1, NUM_EXPERTS), lambda e: (0, 0)),
            pl.BlockSpec((1, H, F), lambda e: (e, 0, 0)),
            pl.BlockSpec((1, F, H), lambda e: (e, 0, 0)),
        ],
        out_specs=pl.BlockSpec((N, H), lambda e: (0, 0)),
        out_shape=jax.ShapeDtypeStruct((N, H), jnp.float32),
        scratch_shapes=[pltpu.VMEM((N, NUM_EXPERTS), jnp.float32)],
    )(x_flat, router_w, rb, w1, w2)
    return out.reshape(B, T, H)
